# baseline (speedup 1.0000x reference)
import functools
import math

import jax
import jax.numpy as jnp
from jax import lax
from jax.experimental import pallas as pl
from jax.experimental.pallas import tpu as pltpu
from jax.experimental.pallas import tpu_sc as plsc

_VOCAB = 1000000
_D = 64
_MAX_LEN = 50
_BATCH = 4096

_NC = 2
_NS = 16
_NW = _NC * _NS
_ROWS = _BATCH * _MAX_LEN
_ROWS_PER_W = _ROWS // _NW
_SEQ_PER_CHUNK = 2
_CHUNK = _SEQ_PER_CHUNK * _MAX_LEN
_CHUNKS_PER_W = _ROWS_PER_W // _CHUNK
_PAIRS = _CHUNKS_PER_W // 2
_SCALE = math.sqrt(_D)
_LANES = 16
_SUBV = _D // _LANES


def _pe_table():
    pos = jnp.arange(_MAX_LEN, dtype=jnp.float32)[:, None]
    i_even = jnp.arange(0, _D, 2, dtype=jnp.float32)[None, :]
    sin_part = jnp.sin(pos / jnp.power(10000.0, 2.0 * i_even / _D))
    cos_part = jnp.cos(pos / jnp.power(10000.0, 2.0 * (i_even + 1.0) / _D))
    pe = jnp.zeros((_MAX_LEN, _D), dtype=jnp.float32)
    pe = pe.at[:, 0::2].set(sin_part)
    pe = pe.at[:, 1::2].set(cos_part)
    return pe


_mesh = plsc.VectorSubcoreMesh(core_axis_name="c", subcore_axis_name="s")


@functools.partial(
    pl.kernel,
    mesh=_mesh,
    out_type=jax.ShapeDtypeStruct((_ROWS, _D), jnp.float32),
    compiler_params=pltpu.CompilerParams(use_tc_tiling_on_sc=False),
    scratch_types=[
        pltpu.VMEM((_CHUNKS_PER_W, _CHUNK), jnp.int32),
        pltpu.VMEM((_CHUNK, _D), jnp.float32),
        pltpu.VMEM((_CHUNK, _D), jnp.float32),
        pltpu.VMEM((_CHUNK, _D), jnp.float32),
        pltpu.SemaphoreType.DMA,
        pltpu.SemaphoreType.DMA,
        pltpu.SemaphoreType.DMA,
        pltpu.SemaphoreType.DMA,
    ],
)
def _sc_embed(idx_hbm, pe_hbm, table_hbm, out_hbm,
              idx_v, pe_v, buf0, buf1, g0, g1, w0, w1):
    wid = lax.axis_index("s") * _NC + lax.axis_index("c")
    chunk0 = wid * _CHUNKS_PER_W
    row0 = wid * _ROWS_PER_W

    pltpu.sync_copy(idx_hbm.at[pl.ds(chunk0, _CHUNKS_PER_W)], idx_v)
    pltpu.sync_copy(pe_hbm, pe_v)

    def gather_start(j, buf, sem):
        pltpu.async_copy(table_hbm.at[idx_v.at[j]], buf, sem)

    def gather_wait(buf, sem):
        pltpu.make_async_copy(table_hbm.at[idx_v.at[0]], buf, sem).wait()

    def wb_start(j, buf, sem):
        pltpu.async_copy(buf, out_hbm.at[pl.ds(row0 + j * _CHUNK, _CHUNK)], sem)

    def wb_wait(buf, sem):
        pltpu.make_async_copy(buf, out_hbm.at[pl.ds(row0, _CHUNK)], sem).wait()

    def compute(buf):
        def rbody(r, c):
            for k in range(2):
                row = 2 * r + k
                for d in range(_SUBV):
                    sl = pl.ds(d * _LANES, _LANES)
                    buf[row, sl] = buf[row, sl] * _SCALE + pe_v[row, sl]
            return c

        lax.fori_loop(0, _CHUNK // 2, rbody, 0)

    gather_start(0, buf0, g0)

    def body(g, c):
        c0 = 2 * g
        c1 = 2 * g + 1

        @pl.when(g > 0)
        def _():
            wb_wait(buf1, w1)

        gather_start(c1, buf1, g1)
        gather_wait(buf0, g0)
        compute(buf0)
        wb_start(c0, buf0, w0)

        @pl.when(g + 1 < _PAIRS)
        def _():
            wb_wait(buf0, w0)
            gather_start(c0 + 2, buf0, g0)

        gather_wait(buf1, g1)
        compute(buf1)
        wb_start(c1, buf1, w1)
        return c

    lax.fori_loop(0, _PAIRS, body, 0)

    wb_wait(buf0, w0)
    wb_wait(buf1, w1)


def kernel(encoded_words, embed_weight):
    idx = encoded_words.astype(jnp.int32).reshape(_ROWS // _CHUNK, _CHUNK)
    pe = _pe_table()
    pe2 = jnp.concatenate([pe] * _SEQ_PER_CHUNK, axis=0)
    out = _sc_embed(idx, pe2, embed_weight)
    return out.reshape(_BATCH, _MAX_LEN, _D)
